# baseline (speedup 1.0000x reference)
import functools

import jax
import jax.numpy as jnp
from jax import lax
from jax.experimental import pallas as pl
from jax.experimental.pallas import tpu as pltpu
from jax.experimental.pallas import tpu_sc as plsc

N_USERS = 100000
N_ITEMS = 1000000
EMB_DIM = 32
BATCH = 16384

_NC = 2
_NS = 16
_NW = _NC * _NS
_BPW = BATCH // _NW
_CHK = 256
_NTASK = 3 * (_BPW // _CHK)


def _select(idx_ref, c, w_ref, o_ref):
    def body(g, carry):
        jv = lax.iota(jnp.int32, 16) + g * 16
        ich = idx_ref[pl.ds(c * _CHK + g * 16, 16)]
        colb = (ich & 3) << 5
        orow = jv >> 2
        ocolb = (jv & 3) << 5
        for d in range(EMB_DIM):
            val = plsc.load_gather(w_ref, [jv, colb + d])
            plsc.store_scatter(o_ref, [orow, ocolb + d], val)
        return carry

    lax.fori_loop(0, _CHK // 16, body, 0)


def _gather3(uids, iids1, iids2, utab, itab, uout, i1out, i2out,
             idx_u, idx_1, idx_2,
             blk0, blk1, blk2, w0, w1, w2, o0, o1,
             g0, g1, g2, ws0, ws1):
    wid = lax.axis_index("s") * _NC + lax.axis_index("c")
    base = wid * _BPW

    pltpu.sync_copy(uids.at[pl.ds(base, _BPW)], idx_u)
    pltpu.sync_copy(iids1.at[pl.ds(base, _BPW)], idx_1)
    pltpu.sync_copy(iids2.at[pl.ds(base, _BPW)], idx_2)

    tasks = [(idx_u, utab, uout, 0), (idx_u, utab, uout, 1),
             (idx_1, itab, i1out, 0), (idx_1, itab, i1out, 1),
             (idx_2, itab, i2out, 0), (idx_2, itab, i2out, 1)]
    blk = [blk0, blk1, blk2]
    wide = [w0, w1, w2]
    gsem = [g0, g1, g2]
    outb = [o0, o1]
    wsem = [ws0, ws1]
    copies = [None, None, None]
    wbs = [None, None]

    def prep_fire(k):
        idx_r, tab, _, c = tasks[k]
        s = k % 3
        for i in range(_CHK // 16):
            blk[s][pl.ds(i * 16, 16)] = (
                idx_r[pl.ds(c * _CHK + i * 16, 16)] >> 2)
        copies[s] = pltpu.async_copy(tab.at[blk[s]], wide[s], gsem[s])

    prep_fire(0)
    prep_fire(1)
    prep_fire(2)
    for k in range(_NTASK):
        s = k % 3
        o = k % 2
        idx_r, _, out_r, c = tasks[k]
        copies[s].wait()
        if wbs[o] is not None:
            wbs[o].wait()
        _select(idx_r, c, wide[s], outb[o])
        orow = pl.multiple_of((base + c * _CHK) // 4, _CHK // 4)
        wbs[o] = pltpu.async_copy(
            outb[o], out_r.at[pl.ds(orow, _CHK // 4)], wsem[o])
        if k + 3 < _NTASK:
            prep_fire(k + 3)
    wbs[0].wait()
    wbs[1].wait()


@jax.jit
def _run(user_ids, item_ids_1, item_ids_2, user_emb, item_emb):
    mesh = plsc.VectorSubcoreMesh(core_axis_name="c", subcore_axis_name="s")
    f32 = jnp.float32
    i32 = jnp.int32
    utab = user_emb.reshape(N_USERS // 4, 128)
    itab = item_emb.reshape(N_ITEMS // 4, 128)
    call = functools.partial(
        pl.kernel,
        mesh=mesh,
        compiler_params=pltpu.CompilerParams(needs_layout_passes=False),
        out_type=(
            jax.ShapeDtypeStruct((BATCH // 4, 128), f32),
            jax.ShapeDtypeStruct((BATCH // 4, 128), f32),
            jax.ShapeDtypeStruct((BATCH // 4, 128), f32),
        ),
        scratch_types=[
            pltpu.VMEM((_BPW,), i32),
            pltpu.VMEM((_BPW,), i32),
            pltpu.VMEM((_BPW,), i32),
            pltpu.VMEM((_CHK,), i32),
            pltpu.VMEM((_CHK,), i32),
            pltpu.VMEM((_CHK,), i32),
            pltpu.VMEM((_CHK, 128), f32),
            pltpu.VMEM((_CHK, 128), f32),
            pltpu.VMEM((_CHK, 128), f32),
            pltpu.VMEM((_CHK // 4, 128), f32),
            pltpu.VMEM((_CHK // 4, 128), f32),
            pltpu.SemaphoreType.DMA,
            pltpu.SemaphoreType.DMA,
            pltpu.SemaphoreType.DMA,
            pltpu.SemaphoreType.DMA,
            pltpu.SemaphoreType.DMA,
        ],
    )(_gather3)
    uw, i1w, i2w = call(user_ids.astype(i32), item_ids_1, item_ids_2,
                        utab, itab)
    shp = (BATCH, EMB_DIM)
    return (uw.reshape(shp), i1w.reshape(shp), i2w.reshape(shp))


def kernel(user_ids, item_ids_1, item_ids_2, user_emb, item_emb):
    return _run(user_ids, item_ids_1, item_ids_2, user_emb, item_emb)
